# baseline (speedup 1.0000x reference)
import numpy as np
import jax
import jax.numpy as jnp
from jax import lax
from jax.experimental import pallas as pl
from jax.experimental.pallas import tpu as pltpu

_B = 8

def _pmap(p, k):
    y = p + (k - 1)
    pp = y % 2
    return pp, (y - pp) // 2


_ROWSETS = {0: (0, 1), 1: (-1, 0)}

_FRAMES16 = [(pp, qq, di, dj)
             for pp in (0, 1) for qq in (0, 1)
             for di in _ROWSETS[pp] for dj in _ROWSETS[qq]]
_FRAMES9 = [(di, dj) for di in (-1, 0, 1) for dj in (-1, 0, 1)]


def _np_consts(H):
    S = H * H
    S1 = np.zeros((2, 2, 16, 3, 3), np.float32)
    for p in (0, 1):
        for q in (0, 1):
            for f, (pp, qq, di, dj) in enumerate(_FRAMES16):
                for kh in range(3):
                    for kw in range(3):
                        if _pmap(p, kh) == (pp, di) and _pmap(q, kw) == (qq, dj):
                            S1[p, q, f, kh, kw] = 1.0
    R = np.zeros((2, 3, 3, 2), np.float32)
    for p in (0, 1):
        for kh in range(3):
            pp, di = _pmap(p, kh)
            R[p, di + 1, kh, pp] = 1.0
    V = np.zeros((2, 3, H), np.float32)
    for p in (0, 1):
        for kh in range(3):
            _, di = _pmap(p, kh)
            for i in range(H):
                if 0 <= i + di < H:
                    V[p, kh, i] = 1.0
    masks9 = np.zeros((9, S), np.float32)
    for t, (di, dj) in enumerate(_FRAMES9):
        for i in range(H):
            for j in range(H):
                if 0 <= i + di < H and 0 <= j + dj < H:
                    masks9[t, i * H + j] = 1.0
    Esplit = np.zeros((4 * S, 4 * S), np.float32)
    for p in (0, 1):
        for q in (0, 1):
            for i in range(H):
                for j in range(H):
                    m = (2 * i + p) * 2 * H + (2 * j + q)
                    Esplit[m, (2 * p + q) * S + i * H + j] = 1.0
    Emerge = Esplit.T.copy()
    return S1, R, V, masks9, Esplit, Emerge


def kernel(up_w, up_b, c1_w, c2_w, bn1_g, bn1_b, bn2_g, bn2_b, x1, x2):
    x1 = x1.astype(jnp.float32)
    x2 = x2.astype(jnp.float32)
    N, Cin, H, W = x1.shape
    _, Ch, H2, W2 = x2.shape
    Cmid = c1_w.shape[0]
    Cout = c2_w.shape[0]
    S = H * W
    M = H2 * W2
    B = _B
    G = N // B
    BS = B * S
    R4 = 4 * Cmid
    eps = 1e-5
    f32, bf16 = jnp.float32, jnp.bfloat16

    S1n, Rn, Vn, m9n, Esn, Emn = _np_consts(H)
    S1 = jnp.asarray(S1n)
    R = jnp.asarray(Rn)
    V = jnp.asarray(Vn)
    masks = jnp.asarray(m9n).astype(bf16)
    Esplit = jnp.asarray(Esn).astype(bf16)
    Emerge = jnp.asarray(Emn).astype(bf16)

    up4 = jnp.transpose(up_w, (2, 3, 1, 0))
    c1a = c1_w[:, :Ch]
    c1b = c1_w[:, Ch:]
    W1a = jnp.einsum('pqfhw,ochw->pqofc', S1, c1a).reshape(R4, 16 * Ch)
    Wc = jnp.einsum('pahm,qbwn,ochw,mnci->pqoabi',
                    R, R, c1b, up4).reshape(R4, 9 * Cin)
    W1S = jnp.concatenate([W1a, Wc], axis=1).astype(bf16)
    W2S = jnp.einsum('pqfhw,ochw->pqofc', S1, c2_w).reshape(
        R4, 16 * Cmid).astype(bf16)
    bvec = jnp.einsum('ochw,c->ohw', c1b, up_b)
    bias1 = jnp.einsum('phi,qwj,ohw->pqoij', V, V, bvec).reshape(R4, S)
    bias1 = jnp.tile(bias1, (1, B))
    g1t = jnp.tile(bn1_g, 4).reshape(R4, 1)
    be1t = jnp.tile(bn1_b, 4).reshape(R4, 1)
    g2t = jnp.tile(bn2_g, 4).reshape(R4, 1)
    be2t = jnp.tile(bn2_b, 4).reshape(R4, 1)

    x1f = x1.reshape(N, Cin, S).astype(bf16)
    x2f = x2.reshape(N, Ch, M).astype(bf16)

    K1 = 16 * Ch + 9 * Cin
    K2 = 16 * Cmid
    cnt = float(N * M)

    def mega(x2_ref, x1_ref, es_ref, w1_ref, bias_ref, mk_ref, w2_ref,
             em_ref, g1_ref, be1_ref, g2_ref, be2_ref,
             out_ref, h1s, h2s, slab_ref, st_ref, ab_ref):
        g = pl.program_id(0)
        mk = mk_ref[...]

        def frames16(src, row_h, lane_ofs):
            for f, (pp, qq, di, dj) in enumerate(_FRAMES16):
                off = di * H + dj
                t9 = (di + 1) * 3 + (dj + 1)
                blk = src[(2 * pp + qq) * row_h:(2 * pp + qq + 1) * row_h, :]
                sh = blk if off == 0 else pltpu.roll(
                    blk, shift=(-off) % S, axis=1)
                slab_ref[f * row_h:(f + 1) * row_h,
                         lane_ofs:lane_ofs + S] = sh * mk[t9:t9 + 1, :]

        @pl.when(g == 0)
        def _init():
            st_ref[...] = jnp.zeros((R4, 128), f32)

        @pl.when(g < G)
        def _phase1():
            x2all = x2_ref[...].reshape(B * Ch, M)
            planes = jnp.dot(x2all, es_ref[...],
                             preferred_element_type=f32).astype(bf16)
            for b in range(B):
                for f, (pp, qq, di, dj) in enumerate(_FRAMES16):
                    off = di * H + dj
                    t9 = (di + 1) * 3 + (dj + 1)
                    blk = planes[b * Ch:(b + 1) * Ch,
                                 (2 * pp + qq) * S:(2 * pp + qq + 1) * S]
                    sh = blk if off == 0 else pltpu.roll(
                        blk, shift=(-off) % S, axis=1)
                    slab_ref[f * Ch:(f + 1) * Ch,
                             b * S:(b + 1) * S] = sh * mk[t9:t9 + 1, :]
                x1b = x1_ref[b]
                base = 16 * Ch
                for t, (di, dj) in enumerate(_FRAMES9):
                    off = di * H + dj
                    t9 = (di + 1) * 3 + (dj + 1)
                    sh = x1b if off == 0 else pltpu.roll(
                        x1b, shift=(-off) % S, axis=1)
                    slab_ref[base + t * Cin:base + (t + 1) * Cin,
                             b * S:(b + 1) * S] = sh * mk[t9:t9 + 1, :]
            acc = jnp.dot(w1_ref[...], slab_ref[...],
                          preferred_element_type=f32) + bias_ref[...]
            h1s[g] = acc.astype(bf16)
            st_ref[:, 0:1] = st_ref[:, 0:1] + jnp.sum(acc, 1, keepdims=True)
            st_ref[:, 1:2] = st_ref[:, 1:2] + jnp.sum(acc * acc, 1,
                                                      keepdims=True)

        def bn_scalars(col, g_ref, be_ref, ocol):
            s4 = (st_ref[0:Cmid, col:col + 1]
                  + st_ref[Cmid:2 * Cmid, col:col + 1]
                  + st_ref[2 * Cmid:3 * Cmid, col:col + 1]
                  + st_ref[3 * Cmid:4 * Cmid, col:col + 1])
            q4 = (st_ref[0:Cmid, col + 1:col + 2]
                  + st_ref[Cmid:2 * Cmid, col + 1:col + 2]
                  + st_ref[2 * Cmid:3 * Cmid, col + 1:col + 2]
                  + st_ref[3 * Cmid:4 * Cmid, col + 1:col + 2])
            m = s4 / cnt
            v = jnp.maximum(q4 / cnt - m * m, 0.0)
            a = g_ref[0:Cmid] * lax.rsqrt(v + eps)
            b = be_ref[0:Cmid] - m * a
            for blk in range(4):
                ab_ref[blk * Cmid:(blk + 1) * Cmid, ocol:ocol + 1] = a
                ab_ref[blk * Cmid:(blk + 1) * Cmid, ocol + 1:ocol + 2] = b

        @pl.when(g == G)
        def _bn1():
            bn_scalars(0, g1_ref, be1_ref, 0)

        @pl.when((g >= G) & (g < 2 * G))
        def _phase2():
            i = g - G
            act = jnp.maximum(h1s[i].astype(f32) * ab_ref[:, 0:1]
                              + ab_ref[:, 1:2], 0.0).astype(bf16)
            for b in range(B):
                frames16(act[:, b * S:(b + 1) * S], Cmid, b * S)
            acc = jnp.dot(w2_ref[...], slab_ref[0:K2, :],
                          preferred_element_type=f32)
            h2s[i] = acc.astype(bf16)
            st_ref[:, 2:3] = st_ref[:, 2:3] + jnp.sum(acc, 1, keepdims=True)
            st_ref[:, 3:4] = st_ref[:, 3:4] + jnp.sum(acc * acc, 1,
                                                      keepdims=True)

        @pl.when(g == 2 * G)
        def _bn2():
            bn_scalars(2, g2_ref, be2_ref, 2)

        @pl.when(g >= 2 * G)
        def _phase3():
            i = g - 2 * G
            act = jnp.maximum(h2s[i].astype(f32) * ab_ref[:, 2:3]
                              + ab_ref[:, 3:4], 0.0).astype(bf16)
            for b in range(B):
                a = jnp.concatenate(
                    [act[pq * Cout:(pq + 1) * Cout, b * S:(b + 1) * S]
                     for pq in range(4)], axis=1)
                out_ref[b] = jnp.dot(a, em_ref[...],
                                     preferred_element_type=f32)

    out = pl.pallas_call(
        mega,
        grid=(3 * G,),
        in_specs=[
            pl.BlockSpec((B, Ch, M), lambda g: (jnp.minimum(g, G - 1), 0, 0)),
            pl.BlockSpec((B, Cin, S), lambda g: (jnp.minimum(g, G - 1), 0, 0)),
            pl.BlockSpec((M, 4 * S), lambda g: (0, 0)),
            pl.BlockSpec((R4, K1), lambda g: (0, 0)),
            pl.BlockSpec((R4, BS), lambda g: (0, 0)),
            pl.BlockSpec((9, S), lambda g: (0, 0)),
            pl.BlockSpec((R4, K2), lambda g: (0, 0)),
            pl.BlockSpec((4 * S, M), lambda g: (0, 0)),
            pl.BlockSpec((R4, 1), lambda g: (0, 0)),
            pl.BlockSpec((R4, 1), lambda g: (0, 0)),
            pl.BlockSpec((R4, 1), lambda g: (0, 0)),
            pl.BlockSpec((R4, 1), lambda g: (0, 0)),
        ],
        out_specs=pl.BlockSpec(
            (B, Cout, M), lambda g: (jnp.maximum(g - 2 * G, 0), 0, 0)),
        out_shape=jax.ShapeDtypeStruct((N, Cout, M), f32),
        scratch_shapes=[
            pltpu.VMEM((G, R4, BS), bf16),
            pltpu.VMEM((G, R4, BS), bf16),
            pltpu.VMEM((K1, BS), bf16),
            pltpu.VMEM((R4, 128), f32),
            pltpu.VMEM((R4, 128), f32),
        ],
        compiler_params=pltpu.CompilerParams(
            dimension_semantics=("arbitrary",),
            vmem_limit_bytes=100 * 1024 * 1024,
        ),
    )(x2f, x1f, Esplit, W1S, bias1, masks, W2S, Emerge,
      g1t, be1t, g2t, be2t)

    return out.reshape(N, Cout, H2, W2)


# speedup vs baseline: 1.0739x; 1.0739x over previous
import numpy as np
import jax
import jax.numpy as jnp
from jax import lax
from jax.experimental import pallas as pl
from jax.experimental.pallas import tpu as pltpu

_B = 8

def _pmap(p, k):
    y = p + (k - 1)
    pp = y % 2
    return pp, (y - pp) // 2


_ROWSETS = {0: (0, 1), 1: (-1, 0)}

_FRAMES16 = [(pp, qq, di, dj)
             for pp in (0, 1) for qq in (0, 1)
             for di in _ROWSETS[pp] for dj in _ROWSETS[qq]]
_FRAMES9 = [(di, dj) for di in (-1, 0, 1) for dj in (-1, 0, 1)]


def _np_consts(H):
    S = H * H
    S1 = np.zeros((2, 2, 16, 3, 3), np.float32)
    for p in (0, 1):
        for q in (0, 1):
            for f, (pp, qq, di, dj) in enumerate(_FRAMES16):
                for kh in range(3):
                    for kw in range(3):
                        if _pmap(p, kh) == (pp, di) and _pmap(q, kw) == (qq, dj):
                            S1[p, q, f, kh, kw] = 1.0
    R = np.zeros((2, 3, 3, 2), np.float32)
    for p in (0, 1):
        for kh in range(3):
            pp, di = _pmap(p, kh)
            R[p, di + 1, kh, pp] = 1.0
    V = np.zeros((2, 3, H), np.float32)
    for p in (0, 1):
        for kh in range(3):
            _, di = _pmap(p, kh)
            for i in range(H):
                if 0 <= i + di < H:
                    V[p, kh, i] = 1.0
    masks9 = np.zeros((9, S), np.float32)
    for t, (di, dj) in enumerate(_FRAMES9):
        for i in range(H):
            for j in range(H):
                if 0 <= i + di < H and 0 <= j + dj < H:
                    masks9[t, i * H + j] = 1.0
    Esplit = np.zeros((4 * S, 4 * S), np.float32)
    for p in (0, 1):
        for q in (0, 1):
            for i in range(H):
                for j in range(H):
                    m = (2 * i + p) * 2 * H + (2 * j + q)
                    Esplit[m, (2 * p + q) * S + i * H + j] = 1.0
    Emerge = Esplit.T.copy()
    return S1, R, V, masks9, Esplit, Emerge


def kernel(up_w, up_b, c1_w, c2_w, bn1_g, bn1_b, bn2_g, bn2_b, x1, x2):
    x1 = x1.astype(jnp.float32)
    x2 = x2.astype(jnp.float32)
    N, Cin, H, W = x1.shape
    _, Ch, H2, W2 = x2.shape
    Cmid = c1_w.shape[0]
    Cout = c2_w.shape[0]
    S = H * W
    M = H2 * W2
    B = _B
    G = N // B
    BS = B * S
    R4 = 4 * Cmid
    eps = 1e-5
    f32, bf16 = jnp.float32, jnp.bfloat16

    S1n, Rn, Vn, m9n, Esn, Emn = _np_consts(H)
    S1 = jnp.asarray(S1n)
    R = jnp.asarray(Rn)
    V = jnp.asarray(Vn)
    masks = jnp.asarray(m9n).astype(bf16)
    Esplit = jnp.asarray(Esn).astype(bf16)
    Emerge = jnp.asarray(Emn).astype(bf16)

    up4 = jnp.transpose(up_w, (2, 3, 1, 0))
    c1a = c1_w[:, :Ch]
    c1b = c1_w[:, Ch:]
    W1a = jnp.einsum('pqfhw,ochw->pqofc', S1, c1a).reshape(R4, 16 * Ch)
    Wc = jnp.einsum('pahm,qbwn,ochw,mnci->pqoabi',
                    R, R, c1b, up4).reshape(R4, 9 * Cin)
    W1S = jnp.concatenate([W1a, Wc], axis=1).astype(bf16)
    W2S = jnp.einsum('pqfhw,ochw->pqofc', S1, c2_w).reshape(
        R4, 16 * Cmid).astype(bf16)
    bvec = jnp.einsum('ochw,c->ohw', c1b, up_b)
    bias1 = jnp.einsum('phi,qwj,ohw->pqoij', V, V, bvec).reshape(R4, S)
    bias1 = jnp.tile(bias1, (1, B))
    g1t = jnp.tile(bn1_g, 4).reshape(R4, 1)
    be1t = jnp.tile(bn1_b, 4).reshape(R4, 1)
    g2t = jnp.tile(bn2_g, 4).reshape(R4, 1)
    be2t = jnp.tile(bn2_b, 4).reshape(R4, 1)

    x1f = x1.reshape(N, Cin, S)
    x2f = x2.reshape(N, Ch, M)

    K1 = 16 * Ch + 9 * Cin
    K2 = 16 * Cmid
    cnt = float(N * M)

    def mega(x2_ref, x1_ref, es_ref, w1_ref, bias_ref, mk_ref, w2_ref,
             em_ref, g1_ref, be1_ref, g2_ref, be2_ref,
             out_ref, h1s, h2s, slab_ref, st_ref, ab_ref):
        g = pl.program_id(0)
        mk = mk_ref[...]

        def frames16(src, row_h, lane_ofs):
            for f, (pp, qq, di, dj) in enumerate(_FRAMES16):
                off = di * H + dj
                t9 = (di + 1) * 3 + (dj + 1)
                blk = src[(2 * pp + qq) * row_h:(2 * pp + qq + 1) * row_h, :]
                sh = blk if off == 0 else pltpu.roll(
                    blk, shift=(-off) % S, axis=1)
                slab_ref[f * row_h:(f + 1) * row_h,
                         lane_ofs:lane_ofs + S] = sh * mk[t9:t9 + 1, :]

        @pl.when(g == 0)
        def _init():
            st_ref[...] = jnp.zeros((R4, 128), f32)

        @pl.when(g < G)
        def _phase1():
            x2all = x2_ref[...].reshape(B * Ch, M).astype(bf16)
            planes = jnp.dot(x2all, es_ref[...],
                             preferred_element_type=f32).astype(bf16)
            for b in range(B):
                for f, (pp, qq, di, dj) in enumerate(_FRAMES16):
                    off = di * H + dj
                    t9 = (di + 1) * 3 + (dj + 1)
                    blk = planes[b * Ch:(b + 1) * Ch,
                                 (2 * pp + qq) * S:(2 * pp + qq + 1) * S]
                    sh = blk if off == 0 else pltpu.roll(
                        blk, shift=(-off) % S, axis=1)
                    slab_ref[f * Ch:(f + 1) * Ch,
                             b * S:(b + 1) * S] = sh * mk[t9:t9 + 1, :]
                x1b = x1_ref[b].astype(bf16)
                base = 16 * Ch
                for t, (di, dj) in enumerate(_FRAMES9):
                    off = di * H + dj
                    t9 = (di + 1) * 3 + (dj + 1)
                    sh = x1b if off == 0 else pltpu.roll(
                        x1b, shift=(-off) % S, axis=1)
                    slab_ref[base + t * Cin:base + (t + 1) * Cin,
                             b * S:(b + 1) * S] = sh * mk[t9:t9 + 1, :]
            acc = jnp.dot(w1_ref[...], slab_ref[...],
                          preferred_element_type=f32) + bias_ref[...]
            h1s[g] = acc.astype(bf16)
            st_ref[:, 0:1] = st_ref[:, 0:1] + jnp.sum(acc, 1, keepdims=True)
            st_ref[:, 1:2] = st_ref[:, 1:2] + jnp.sum(acc * acc, 1,
                                                      keepdims=True)

        def bn_scalars(col, g_ref, be_ref, ocol):
            s4 = (st_ref[0:Cmid, col:col + 1]
                  + st_ref[Cmid:2 * Cmid, col:col + 1]
                  + st_ref[2 * Cmid:3 * Cmid, col:col + 1]
                  + st_ref[3 * Cmid:4 * Cmid, col:col + 1])
            q4 = (st_ref[0:Cmid, col + 1:col + 2]
                  + st_ref[Cmid:2 * Cmid, col + 1:col + 2]
                  + st_ref[2 * Cmid:3 * Cmid, col + 1:col + 2]
                  + st_ref[3 * Cmid:4 * Cmid, col + 1:col + 2])
            m = s4 / cnt
            v = jnp.maximum(q4 / cnt - m * m, 0.0)
            a = g_ref[0:Cmid] * lax.rsqrt(v + eps)
            b = be_ref[0:Cmid] - m * a
            for blk in range(4):
                ab_ref[blk * Cmid:(blk + 1) * Cmid, ocol:ocol + 1] = a
                ab_ref[blk * Cmid:(blk + 1) * Cmid, ocol + 1:ocol + 2] = b

        @pl.when(g == G)
        def _bn1():
            bn_scalars(0, g1_ref, be1_ref, 0)

        @pl.when((g >= G) & (g < 2 * G))
        def _phase2():
            i = g - G
            act = jnp.maximum(h1s[i].astype(f32) * ab_ref[:, 0:1]
                              + ab_ref[:, 1:2], 0.0).astype(bf16)
            for b in range(B):
                frames16(act[:, b * S:(b + 1) * S], Cmid, b * S)
            acc = jnp.dot(w2_ref[...], slab_ref[0:K2, :],
                          preferred_element_type=f32)
            h2s[i] = acc.astype(bf16)
            st_ref[:, 2:3] = st_ref[:, 2:3] + jnp.sum(acc, 1, keepdims=True)
            st_ref[:, 3:4] = st_ref[:, 3:4] + jnp.sum(acc * acc, 1,
                                                      keepdims=True)

        @pl.when(g == 2 * G)
        def _bn2():
            bn_scalars(2, g2_ref, be2_ref, 2)

        @pl.when(g >= 2 * G)
        def _phase3():
            i = g - 2 * G
            act = jnp.maximum(h2s[i].astype(f32) * ab_ref[:, 2:3]
                              + ab_ref[:, 3:4], 0.0).astype(bf16)
            for b in range(B):
                a = jnp.concatenate(
                    [act[pq * Cout:(pq + 1) * Cout, b * S:(b + 1) * S]
                     for pq in range(4)], axis=1)
                out_ref[b] = jnp.dot(a, em_ref[...],
                                     preferred_element_type=f32)

    out = pl.pallas_call(
        mega,
        grid=(3 * G,),
        in_specs=[
            pl.BlockSpec((B, Ch, M), lambda g: (jnp.minimum(g, G - 1), 0, 0)),
            pl.BlockSpec((B, Cin, S), lambda g: (jnp.minimum(g, G - 1), 0, 0)),
            pl.BlockSpec((M, 4 * S), lambda g: (0, 0)),
            pl.BlockSpec((R4, K1), lambda g: (0, 0)),
            pl.BlockSpec((R4, BS), lambda g: (0, 0)),
            pl.BlockSpec((9, S), lambda g: (0, 0)),
            pl.BlockSpec((R4, K2), lambda g: (0, 0)),
            pl.BlockSpec((4 * S, M), lambda g: (0, 0)),
            pl.BlockSpec((R4, 1), lambda g: (0, 0)),
            pl.BlockSpec((R4, 1), lambda g: (0, 0)),
            pl.BlockSpec((R4, 1), lambda g: (0, 0)),
            pl.BlockSpec((R4, 1), lambda g: (0, 0)),
        ],
        out_specs=pl.BlockSpec(
            (B, Cout, M), lambda g: (jnp.maximum(g - 2 * G, 0), 0, 0)),
        out_shape=jax.ShapeDtypeStruct((N, Cout, M), f32),
        scratch_shapes=[
            pltpu.VMEM((G, R4, BS), bf16),
            pltpu.VMEM((G, R4, BS), bf16),
            pltpu.VMEM((K1, BS), bf16),
            pltpu.VMEM((R4, 128), f32),
            pltpu.VMEM((R4, 128), f32),
        ],
        compiler_params=pltpu.CompilerParams(
            dimension_semantics=("arbitrary",),
            vmem_limit_bytes=100 * 1024 * 1024,
        ),
    )(x2f, x1f, Esplit, W1S, bias1, masks, W2S, Emerge,
      g1t, be1t, g2t, be2t)

    return out.reshape(N, Cout, H2, W2)
